# baseline (speedup 1.0000x reference)
import functools
import math

import jax
import jax.numpy as jnp
from jax.experimental import pallas as pl
from jax.experimental.pallas import tpu as pltpu

B, T, D = 1, 2048, 768
E, K = 16, 1
HID = 2048
HEADS = 4
DH = D // HEADS
STEPS = 2
N = B * T
NK = N * K
CAP = 1 << int(math.ceil(math.log2(math.ceil((NK / E) * 1.25))))

_BT_ROUTER = 512
_BT_COLLAB = 256


def _mmT(a, b):
    return jax.lax.dot_general(
        a, b, (((1,), (1,)), ((), ())), preferred_element_type=jnp.float32
    )


def _router_kernel(x_ref, wg_ref, logits_ref):
    logits_ref[...] = _mmT(x_ref[...], wg_ref[...])


_CHUNK = 128
_NCHPE = CAP // _CHUNK
_NCH = E * _NCHPE


def _expert_kernel(counts_ref, xs_ref, w13_ref, w2_ref, out_ref):
    i = pl.program_id(0)
    e = i // _NCHPE
    c = i % _NCHPE

    @pl.when(c * _CHUNK < counts_ref[e])
    def _():
        xg = xs_ref[...]
        z = (xg + xg).astype(jnp.bfloat16)
        gu = _mmT(z, w13_ref[0].astype(jnp.bfloat16))
        g = gu[:, :HID]
        u = gu[:, HID:]
        su = (jax.nn.silu(g) * u).astype(jnp.bfloat16)
        out_ref[...] = xg + _mmT(su, w2_ref[0].astype(jnp.bfloat16))


def _rms(x, w):
    return x * jax.lax.rsqrt(jnp.mean(x * x, axis=-1, keepdims=True) + 1e-6) * w


def _collab_kernel(es_ref, kept_ref, med_ref, wi_ref, bi_ref, wo_ref, bo_ref,
                   n1_ref, n2_ref, w1_ref, w2_ref, fw_ref, fb_ref, wop_ref,
                   out_ref):
    bt = es_ref.shape[0]
    s1 = kept_ref[...] * es_ref[...]
    s0 = jnp.broadcast_to(med_ref[...], (bt, D))
    n1 = n1_ref[...]
    n2 = n2_ref[...]
    wi = wi_ref[...].astype(jnp.bfloat16)
    bi = bi_ref[...]
    wo = wo_ref[...].astype(jnp.bfloat16)
    bo = bo_ref[...]
    w1 = w1_ref[...].astype(jnp.bfloat16)
    w2 = w2_ref[...].astype(jnp.bfloat16)

    lane = jax.lax.broadcasted_iota(jnp.int32, (1, D), 1)
    head_masks = [
        ((lane >= h * DH) & (lane < (h + 1) * DH)).astype(jnp.float32)
        for h in range(HEADS)
    ]
    inv_sqrt_dh = 1.0 / math.sqrt(DH)

    def hdots(a, b):
        p = a * b
        return [jnp.sum(p * m, axis=-1, keepdims=True) for m in head_masks]

    for _ in range(STEPS):
        h0 = _rms(s0, n1).astype(jnp.bfloat16)
        h1 = _rms(s1, n1).astype(jnp.bfloat16)
        qkv0 = _mmT(h0, wi) + bi
        qkv1 = _mmT(h1, wi) + bi
        q0, k0, v0 = qkv0[:, :D], qkv0[:, D:2 * D], qkv0[:, 2 * D:]
        q1, k1, v1 = qkv1[:, :D], qkv1[:, D:2 * D], qkv1[:, 2 * D:]

        l00 = hdots(q0, k0)
        l01 = hdots(q0, k1)
        l10 = hdots(q1, k0)
        l11 = hdots(q1, k1)

        a00 = jnp.zeros((bt, D), jnp.float32)
        a01 = jnp.zeros((bt, D), jnp.float32)
        a10 = jnp.zeros((bt, D), jnp.float32)
        a11 = jnp.zeros((bt, D), jnp.float32)
        for h in range(HEADS):
            x00 = l00[h] * inv_sqrt_dh
            x01 = l01[h] * inv_sqrt_dh
            x10 = l10[h] * inv_sqrt_dh
            x11 = l11[h] * inv_sqrt_dh
            m0 = jnp.maximum(x00, x01)
            e00 = jnp.exp(x00 - m0)
            e01 = jnp.exp(x01 - m0)
            d0 = e00 + e01
            m1 = jnp.maximum(x10, x11)
            e10 = jnp.exp(x10 - m1)
            e11 = jnp.exp(x11 - m1)
            d1 = e10 + e11
            a00 = a00 + (e00 / d0) * head_masks[h]
            a01 = a01 + (e01 / d0) * head_masks[h]
            a10 = a10 + (e10 / d1) * head_masks[h]
            a11 = a11 + (e11 / d1) * head_masks[h]

        o0 = (a00 * v0 + a01 * v1).astype(jnp.bfloat16)
        o1 = (a10 * v0 + a11 * v1).astype(jnp.bfloat16)
        s0 = s0 + _mmT(o0, wo) + bo
        s1 = s1 + _mmT(o1, wo) + bo

        g0 = _rms(s0, n2).astype(jnp.bfloat16)
        g1 = _rms(s1, n2).astype(jnp.bfloat16)

        def gelu(x):
            return 0.5 * x * (1.0 + jax.lax.erf(x / math.sqrt(2.0)))

        s0 = s0 + _mmT(gelu(_mmT(g0, w1)).astype(jnp.bfloat16), w2)
        s1 = s1 + _mmT(gelu(_mmT(g1, w1)).astype(jnp.bfloat16), w2)

    agg = kept_ref[...] * s1
    gdot = jnp.sum(agg * fw_ref[...], axis=-1, keepdims=True)
    gl = jax.nn.sigmoid(gdot + fb_ref[...])
    fused = (gl * s0 + (1.0 - gl) * agg).astype(jnp.bfloat16)
    out_ref[...] = _mmT(fused, wop_ref[...].astype(jnp.bfloat16))


@jax.jit
def kernel(x, W_gate, w13, w2, mediator, in_proj_w, in_proj_b, out_proj_w,
           out_proj_b, norm1_w, norm2_w, ffn_w1, ffn_w2, fuse_w, fuse_b,
           o_proj_w):
    x_flat = x.reshape(N, D).astype(jnp.float32)

    logits = pl.pallas_call(
        _router_kernel,
        grid=(N // _BT_ROUTER,),
        in_specs=[
            pl.BlockSpec((_BT_ROUTER, D), lambda i: (i, 0)),
            pl.BlockSpec((E, D), lambda i: (0, 0)),
        ],
        out_specs=pl.BlockSpec((_BT_ROUTER, E), lambda i: (i, 0)),
        out_shape=jax.ShapeDtypeStruct((N, E), jnp.float32),
    )(x_flat, W_gate)

    router_probs = jax.nn.softmax(logits, axis=-1)
    tgt = jnp.argmax(logits, axis=-1)
    prio = jnp.max(logits, axis=-1)
    counts = jnp.sum(
        (tgt[:, None] == jnp.arange(E)[None, :]).astype(jnp.float32), axis=0
    )
    balance = jnp.sum(router_probs.mean(axis=0) * counts / N) * E
    z = jax.nn.logsumexp(logits, axis=-1)
    z_loss = jnp.mean(z * z)
    dropped = jnp.sum(jnp.maximum(counts - CAP, 0.0))
    aux = 0.01 * balance + 0.001 * z_loss + 0.001 * dropped / NK

    order = jnp.lexsort((-prio, tgt))
    inv_order = jnp.argsort(order)
    tgt_sorted = tgt[order]
    icounts = counts.astype(jnp.int32)
    starts = jnp.cumsum(icounts) - icounts
    pos = jnp.arange(N, dtype=jnp.int32) - starts[tgt_sorted]
    kept_sorted = (pos < CAP).astype(jnp.float32)
    slot_idx = tgt_sorted * CAP + jnp.clip(pos, 0, CAP - 1)

    slot = jnp.arange(CAP, dtype=jnp.int32)[None, :]
    gpos = starts[:, None] + slot
    layout = order[jnp.clip(gpos, 0, N - 1)].reshape(-1)

    xs = x_flat[layout]

    ys = pl.pallas_call(
        _expert_kernel,
        grid_spec=pltpu.PrefetchScalarGridSpec(
            num_scalar_prefetch=1,
            grid=(_NCH,),
            in_specs=[
                pl.BlockSpec((_CHUNK, D), lambda i, cnt: (i, 0)),
                pl.BlockSpec((1, 2 * HID, D), lambda i, cnt: (i // _NCHPE, 0, 0)),
                pl.BlockSpec((1, D, HID), lambda i, cnt: (i // _NCHPE, 0, 0)),
            ],
            out_specs=pl.BlockSpec((_CHUNK, D), lambda i, cnt: (i, 0)),
        ),
        out_shape=jax.ShapeDtypeStruct((E * CAP, D), jnp.float32),
    )(icounts, xs, w13, w2)

    es_sorted = ys[slot_idx]

    med2 = mediator.reshape(1, D)
    full = lambda r, c: pl.BlockSpec((r, c), lambda i: (0, 0))
    out_flat = pl.pallas_call(
        _collab_kernel,
        grid=(N // _BT_COLLAB,),
        in_specs=[
            pl.BlockSpec((_BT_COLLAB, D), lambda i: (i, 0)),
            pl.BlockSpec((_BT_COLLAB, 1), lambda i: (i, 0)),
            full(1, D),
            full(3 * D, D),
            full(1, 3 * D),
            full(D, D),
            full(1, D),
            full(1, D),
            full(1, D),
            full(D, D),
            full(D, D),
            full(1, D),
            full(1, 1),
            full(D, D),
        ],
        out_specs=pl.BlockSpec((_BT_COLLAB, D), lambda i: (i, 0)),
        out_shape=jax.ShapeDtypeStruct((N, D), jnp.float32),
    )(es_sorted, kept_sorted[:, None], med2, in_proj_w, in_proj_b.reshape(1, -1),
      out_proj_w, out_proj_b.reshape(1, -1), norm1_w.reshape(1, -1),
      norm2_w.reshape(1, -1), ffn_w1, ffn_w2, fuse_w, fuse_b.reshape(1, 1),
      o_proj_w)

    return out_flat[inv_order].reshape(B, T, D), aux, router_probs


# speedup vs baseline: 1.2096x; 1.2096x over previous
import functools
import math

import jax
import jax.numpy as jnp
from jax.experimental import pallas as pl
from jax.experimental.pallas import tpu as pltpu

B, T, D = 1, 2048, 768
E, K = 16, 1
HID = 2048
HEADS = 4
DH = D // HEADS
STEPS = 2
N = B * T
NK = N * K
CAP = 1 << int(math.ceil(math.log2(math.ceil((NK / E) * 1.25))))

_BT_ROUTER = 512
_BT_COLLAB = 512


def _mmT(a, b):
    return jax.lax.dot_general(
        a, b, (((1,), (1,)), ((), ())), preferred_element_type=jnp.float32
    )


def _router_kernel(x_ref, wg_ref, logits_ref):
    logits_ref[...] = _mmT(x_ref[...], wg_ref[...])


def _expert_kernel(xs_ref, w13_ref, w2_ref, out_ref):
    xg = xs_ref[...]
    z = (xg + xg).astype(jnp.bfloat16)
    gu = _mmT(z, w13_ref[0].astype(jnp.bfloat16))
    g = gu[:, :HID]
    u = gu[:, HID:]
    su = (jax.nn.silu(g) * u).astype(jnp.bfloat16)
    out_ref[...] = xg + _mmT(su, w2_ref[0].astype(jnp.bfloat16))


def _rms(x, w):
    return x * jax.lax.rsqrt(jnp.mean(x * x, axis=-1, keepdims=True) + 1e-6) * w


def _collab_kernel(es_ref, kept_ref, med_ref, wi_ref, bi_ref, wo_ref, bo_ref,
                   n1_ref, n2_ref, w1_ref, w2_ref, fw_ref, fb_ref, wop_ref,
                   out_ref):
    bt = es_ref.shape[0]
    s1 = kept_ref[...] * es_ref[...]
    s0 = jnp.broadcast_to(med_ref[...], (bt, D))
    n1 = n1_ref[...]
    n2 = n2_ref[...]
    wi = wi_ref[...].astype(jnp.bfloat16)
    bi = bi_ref[...]
    wo = wo_ref[...].astype(jnp.bfloat16)
    bo = bo_ref[...]
    w1 = w1_ref[...].astype(jnp.bfloat16)
    w2 = w2_ref[...].astype(jnp.bfloat16)

    lane = jax.lax.broadcasted_iota(jnp.int32, (1, D), 1)
    head_masks = [
        ((lane >= h * DH) & (lane < (h + 1) * DH)).astype(jnp.float32)
        for h in range(HEADS)
    ]
    inv_sqrt_dh = 1.0 / math.sqrt(DH)

    def hdots(a, b):
        p = a * b
        return [jnp.sum(p * m, axis=-1, keepdims=True) for m in head_masks]

    for _ in range(STEPS):
        h0 = _rms(s0, n1).astype(jnp.bfloat16)
        h1 = _rms(s1, n1).astype(jnp.bfloat16)
        qkv0 = _mmT(h0, wi) + bi
        qkv1 = _mmT(h1, wi) + bi
        q0, k0, v0 = qkv0[:, :D], qkv0[:, D:2 * D], qkv0[:, 2 * D:]
        q1, k1, v1 = qkv1[:, :D], qkv1[:, D:2 * D], qkv1[:, 2 * D:]

        l00 = hdots(q0, k0)
        l01 = hdots(q0, k1)
        l10 = hdots(q1, k0)
        l11 = hdots(q1, k1)

        a00 = jnp.zeros((bt, D), jnp.float32)
        a01 = jnp.zeros((bt, D), jnp.float32)
        a10 = jnp.zeros((bt, D), jnp.float32)
        a11 = jnp.zeros((bt, D), jnp.float32)
        for h in range(HEADS):
            x00 = l00[h] * inv_sqrt_dh
            x01 = l01[h] * inv_sqrt_dh
            x10 = l10[h] * inv_sqrt_dh
            x11 = l11[h] * inv_sqrt_dh
            m0 = jnp.maximum(x00, x01)
            e00 = jnp.exp(x00 - m0)
            e01 = jnp.exp(x01 - m0)
            d0 = e00 + e01
            m1 = jnp.maximum(x10, x11)
            e10 = jnp.exp(x10 - m1)
            e11 = jnp.exp(x11 - m1)
            d1 = e10 + e11
            a00 = a00 + (e00 / d0) * head_masks[h]
            a01 = a01 + (e01 / d0) * head_masks[h]
            a10 = a10 + (e10 / d1) * head_masks[h]
            a11 = a11 + (e11 / d1) * head_masks[h]

        o0 = (a00 * v0 + a01 * v1).astype(jnp.bfloat16)
        o1 = (a10 * v0 + a11 * v1).astype(jnp.bfloat16)
        s0 = s0 + _mmT(o0, wo) + bo
        s1 = s1 + _mmT(o1, wo) + bo

        g0 = _rms(s0, n2).astype(jnp.bfloat16)
        g1 = _rms(s1, n2).astype(jnp.bfloat16)

        def gelu(x):
            return 0.5 * x * (1.0 + jax.lax.erf(x / math.sqrt(2.0)))

        s0 = s0 + _mmT(gelu(_mmT(g0, w1)).astype(jnp.bfloat16), w2)
        s1 = s1 + _mmT(gelu(_mmT(g1, w1)).astype(jnp.bfloat16), w2)

    agg = kept_ref[...] * s1
    gdot = jnp.sum(agg * fw_ref[...], axis=-1, keepdims=True)
    gl = jax.nn.sigmoid(gdot + fb_ref[...])
    fused = (gl * s0 + (1.0 - gl) * agg).astype(jnp.bfloat16)
    out_ref[...] = _mmT(fused, wop_ref[...].astype(jnp.bfloat16))


@jax.jit
def kernel(x, W_gate, w13, w2, mediator, in_proj_w, in_proj_b, out_proj_w,
           out_proj_b, norm1_w, norm2_w, ffn_w1, ffn_w2, fuse_w, fuse_b,
           o_proj_w):
    x_flat = x.reshape(N, D).astype(jnp.float32)

    logits = pl.pallas_call(
        _router_kernel,
        grid=(N // _BT_ROUTER,),
        in_specs=[
            pl.BlockSpec((_BT_ROUTER, D), lambda i: (i, 0)),
            pl.BlockSpec((E, D), lambda i: (0, 0)),
        ],
        out_specs=pl.BlockSpec((_BT_ROUTER, E), lambda i: (i, 0)),
        out_shape=jax.ShapeDtypeStruct((N, E), jnp.float32),
    )(x_flat, W_gate)

    router_probs = jax.nn.softmax(logits, axis=-1)
    tgt = jnp.argmax(logits, axis=-1)
    prio = jnp.max(logits, axis=-1)
    counts = jnp.sum(
        (tgt[:, None] == jnp.arange(E)[None, :]).astype(jnp.float32), axis=0
    )
    balance = jnp.sum(router_probs.mean(axis=0) * counts / N) * E
    z = jax.nn.logsumexp(logits, axis=-1)
    z_loss = jnp.mean(z * z)
    dropped = jnp.sum(jnp.maximum(counts - CAP, 0.0))
    aux = 0.01 * balance + 0.001 * z_loss + 0.001 * dropped / NK

    order = jnp.lexsort((-prio, tgt))
    inv_order = jnp.argsort(order)
    tgt_sorted = tgt[order]
    icounts = counts.astype(jnp.int32)
    starts = jnp.cumsum(icounts) - icounts
    pos = jnp.arange(N, dtype=jnp.int32) - starts[tgt_sorted]
    kept_sorted = (pos < CAP).astype(jnp.float32)
    slot_idx = tgt_sorted * CAP + jnp.clip(pos, 0, CAP - 1)

    slot = jnp.arange(CAP, dtype=jnp.int32)[None, :]
    gpos = starts[:, None] + slot
    layout = order[jnp.clip(gpos, 0, N - 1)].reshape(-1)

    xs = x_flat[layout]

    ys = pl.pallas_call(
        _expert_kernel,
        grid=(E,),
        in_specs=[
            pl.BlockSpec((CAP, D), lambda e: (e, 0)),
            pl.BlockSpec((1, 2 * HID, D), lambda e: (e, 0, 0)),
            pl.BlockSpec((1, D, HID), lambda e: (e, 0, 0)),
        ],
        out_specs=pl.BlockSpec((CAP, D), lambda e: (e, 0)),
        out_shape=jax.ShapeDtypeStruct((E * CAP, D), jnp.float32),
    )(xs, w13, w2)

    es_sorted = ys[slot_idx]

    med2 = mediator.reshape(1, D)
    full = lambda r, c: pl.BlockSpec((r, c), lambda i: (0, 0))
    out_flat = pl.pallas_call(
        _collab_kernel,
        grid=(N // _BT_COLLAB,),
        in_specs=[
            pl.BlockSpec((_BT_COLLAB, D), lambda i: (i, 0)),
            pl.BlockSpec((_BT_COLLAB, 1), lambda i: (i, 0)),
            full(1, D),
            full(3 * D, D),
            full(1, 3 * D),
            full(D, D),
            full(1, D),
            full(1, D),
            full(1, D),
            full(D, D),
            full(D, D),
            full(1, D),
            full(1, 1),
            full(D, D),
        ],
        out_specs=pl.BlockSpec((_BT_COLLAB, D), lambda i: (i, 0)),
        out_shape=jax.ShapeDtypeStruct((N, D), jnp.float32),
    )(es_sorted, kept_sorted[:, None], med2, in_proj_w, in_proj_b.reshape(1, -1),
      out_proj_w, out_proj_b.reshape(1, -1), norm1_w.reshape(1, -1),
      norm2_w.reshape(1, -1), ffn_w1, ffn_w2, fuse_w, fuse_b.reshape(1, 1),
      o_proj_w)

    return out_flat[inv_order].reshape(B, T, D), aux, router_probs
